# Initial kernel scaffold; baseline (speedup 1.0000x reference)
import jax
import jax.numpy as jnp
from jax.experimental import pallas as pl


def kernel(x):
    raise NotImplementedError("write your pallas kernel here")


# speedup vs baseline: 4.2527x; 4.2527x over previous
import jax
import jax.numpy as jnp
from jax.experimental import pallas as pl

_STRIDE = 16


def _mask_kernel(x_ref, o_ref):
    col = jax.lax.broadcasted_iota(jnp.int32, x_ref.shape, dimension=1)
    keep = (col % _STRIDE) == 0
    o_ref[...] = jnp.where(keep, x_ref[...], 0.0)


def kernel(x):
    m, n = x.shape
    bm = 1024
    return pl.pallas_call(
        _mask_kernel,
        grid=(m // bm,),
        in_specs=[pl.BlockSpec((bm, n), lambda i: (i, 0))],
        out_specs=pl.BlockSpec((bm, n), lambda i: (i, 0)),
        out_shape=jax.ShapeDtypeStruct((m, n), x.dtype),
    )(x)
